# Initial kernel scaffold; baseline (speedup 1.0000x reference)
import jax
import jax.numpy as jnp
from jax.experimental import pallas as pl


def kernel(out0, out1, out2, labels):
    raise NotImplementedError("write your pallas kernel here")


# speedup vs baseline: 36.7093x; 36.7093x over previous
import math

import jax
import jax.numpy as jnp
import numpy as np
from jax import lax
from jax.experimental import pallas as pl
from jax.experimental.pallas import tpu as pltpu

_STRIDES = [8, 16, 32]
_ANCHORS = [[12, 16], [19, 36], [40, 28], [36, 75], [76, 55], [72, 146],
            [142, 110], [192, 243], [459, 401]]
_NCLS = 80
_NCH = 86


def _log2(n):
    return int(n).bit_length() - 1


def _atan(x):
    sgn = jnp.sign(x)
    t = jnp.abs(x)
    big = t > 2.414213562373095
    mid = t > 0.4142135623730951
    safe_t = jnp.where(big, t, 1.0)
    z_arg = jnp.where(big, -1.0 / safe_t,
                      jnp.where(mid, (t - 1.0) / (t + 1.0), t))
    z2 = z_arg * z_arg
    p = (((8.05374449538e-2 * z2 - 1.38776856032e-1) * z2
          + 1.99777106478e-1) * z2 - 3.33329491539e-1) * z2 * z_arg + z_arg
    res = jnp.where(big, math.pi / 2 + p,
                    jnp.where(mid, math.pi / 4 + p, p))
    return sgn * res


def _make_scale_kernel(oid, B, H, W, nmax):
    stride = float(_STRIDES[oid])
    HW = H * W
    P = 3 * HW
    CH = 512 if P % 512 == 0 else 256
    log2_hw = _log2(HW)
    log2_w = _log2(W)
    aw_all = [a[0] / stride for a in _ANCHORS]
    ah_all = [a[1] / stride for a in _ANCHORS]
    atan_ref = [float(np.arctan(np.float32(aw_all[k]) /
                                (np.float32(ah_all[k]) + np.float32(1e-16))))
                for k in range(9)]
    mw = [aw_all[3 * oid + j] for j in range(3)]
    mh = [ah_all[3 * oid + j] for j in range(3)]
    f32 = jnp.float32

    def body(head_ref, full_ref, lab_ref, out_ref, scr_ref):
        x1 = lab_ref[0, 0]
        y1 = lab_ref[0, 1]
        x2 = lab_ref[0, 2]
        y2 = lab_ref[0, 3]
        cls = lab_ref[0, 4]
        dep = lab_ref[0, 5]
        valid = (x1 + y1 + x2 + y2 + cls + dep) > 0.0
        tx = (x2 + x1) / (stride * 2.0)
        ty = (y2 + y1) / (stride * 2.0)
        tw = (x2 - x1) / stride
        th = (y2 - y1) / stride

        area_a = tw * th
        atan_a = _atan(tw / (th + 1e-16))
        best_iou = jnp.full_like(tw, -jnp.inf)
        best_all = jnp.zeros(tw.shape, jnp.int32)
        for k in range(9):
            rw = aw_all[k]
            rh = ah_all[k]
            brx = jnp.minimum(tw, rw)
            bry = jnp.minimum(th, rh)
            en = ((0.0 < brx) & (0.0 < bry)).astype(f32)
            ai = brx * bry * en
            iou = ai / (area_a + (rw * rh) - ai + 1e-16)
            cbx = jnp.maximum(tw, rw)
            cby = jnp.maximum(th, rh)
            c2 = cbx * cbx + cby * cby + 1e-16
            rho2 = (tw - rw) ** 2 / 4.0 + (th - rh) ** 2 / 4.0
            dv = atan_a - atan_ref[k]
            v = (4.0 / math.pi ** 2) * dv * dv
            alpha = v / (1.0 - iou + v + 1e-16)
            ciou = iou - (rho2 / c2 + v * alpha)
            upd = ciou > best_iou
            best_iou = jnp.where(upd, ciou, best_iou)
            best_all = jnp.where(upd, k, best_all)
        best = best_all % 3
        use = valid & ((best_all // 3) == oid)
        has_b = jnp.any(use)

        ti = tx.astype(jnp.int32)
        tj = ty.astype(jnp.int32)
        inb = (ti >= 0) & (tj >= 0) & (ti < W) & (tj < H)
        cond = use & inb
        ic = jnp.clip(ti, 0, W - 1)
        jc = jnp.clip(tj, 0, H - 1)
        key = (best * HW + jc * W + ic).astype(f32)

        key_row = jnp.transpose(key)
        cond_row = jnp.transpose(cond.astype(f32))
        ii = lax.broadcasted_iota(jnp.int32, (nmax, nmax), 0)
        jj = lax.broadcasted_iota(jnp.int32, (nmax, nmax), 1)
        ov = (jj > ii) & (key == key_row) & (cond_row > 0.0)
        overwritten = jnp.any(ov, axis=1, keepdims=True)
        winner = (cond & (~overwritten)).astype(f32)

        ic_f = ic.astype(f32)
        jc_f = jc.astype(f32)
        r0 = tx - ic_f
        r1 = ty - jc_f
        aw_b = jnp.where(best == 0, mw[0], jnp.where(best == 1, mw[1], mw[2]))
        ah_b = jnp.where(best == 0, mh[0], jnp.where(best == 1, mh[1], mh[2]))
        r2 = jnp.log(tw / aw_b + 1e-16)
        r3 = jnp.log(th / ah_b + 1e-16)
        scale_v = jnp.sqrt(jnp.maximum(2.0 - tw * th / float(W * H), 1e-8))

        scr_ref[0:nmax, 0:1] = winner
        scr_ref[0:nmax, 1:2] = best.astype(f32)
        scr_ref[0:nmax, 2:3] = (jc * W + ic).astype(f32)
        scr_ref[0:nmax, 3:4] = ic_f
        scr_ref[0:nmax, 4:5] = jc_f
        scr_ref[0:nmax, 5:6] = r0
        scr_ref[0:nmax, 6:7] = r1
        scr_ref[0:nmax, 7:8] = r2
        scr_ref[0:nmax, 8:9] = r3
        scr_ref[0:nmax, 9:10] = scale_v
        scr_ref[0:nmax, 10:11] = cls
        scr_ref[0:nmax, 11:12] = dep

        c0 = -jnp.log(1.0 - jnp.clip(jnp.zeros((), f32), 1e-7, 1.0 - 1e-7))

        t_lo_x = tx - tw * 0.5
        t_lo_y = ty - th * 0.5
        t_hi_x = tx + tw * 0.5
        t_hi_y = ty + th * 0.5
        twth = tw * th

        acc = jnp.zeros((), f32)
        for cs in range(0, P, CH):
            o0 = head_ref[0, 0, :, cs:cs + CH]
            o1 = head_ref[0, 1, :, cs:cs + CH]
            o2 = head_ref[0, 2, :, cs:cs + CH]
            o3 = head_ref[0, 3, :, cs:cs + CH]
            o4 = head_ref[0, 4, :, cs:cs + CH]
            q = cs + lax.broadcasted_iota(jnp.int32, (1, CH), 1)
            a_idx = q >> log2_hw
            rem = q & (HW - 1)
            gy = (rem >> log2_w).astype(f32)
            gx = (rem & (W - 1)).astype(f32)
            awp = jnp.where(a_idx == 0, mw[0],
                            jnp.where(a_idx == 1, mw[1], mw[2]))
            ahp = jnp.where(a_idx == 0, mh[0],
                            jnp.where(a_idx == 1, mh[1], mh[2]))
            px = jax.nn.sigmoid(o0) + gx
            py = jax.nn.sigmoid(o1) + gy
            pw = jnp.exp(o2) * awp
            ph = jnp.exp(o3) * ahp
            p_lo_x = px - pw * 0.5
            p_lo_y = py - ph * 0.5
            p_hi_x = px + pw * 0.5
            p_hi_y = py + ph * 0.5
            pwph = pw * ph
            tlx = jnp.maximum(p_lo_x, t_lo_x)
            tly = jnp.maximum(p_lo_y, t_lo_y)
            brx = jnp.minimum(p_hi_x, t_hi_x)
            bry = jnp.minimum(p_hi_y, t_hi_y)
            en = ((tlx < brx) & (tly < bry)).astype(f32)
            ai = (brx - tlx) * (bry - tly) * en
            u = (pwph + twth - ai) + 1e-16
            hit = valid & (ai > 0.5 * u)
            pb = jnp.any(hit, axis=0, keepdims=True)
            om = jnp.where(has_b, 1.0 - pb.astype(f32), 1.0)
            p4 = jnp.clip(jax.nn.sigmoid(o4) * om, 1e-7, 1.0 - 1e-7)
            acc = acc + jnp.sum(-jnp.log(1.0 - p4))
        acc = acc + c0 * float(_NCLS * P)

        def corr(t, a):
            row = scr_ref[pl.ds(t, 1), :]
            wflag = row[0, 0]
            a_i = row[0, 1].astype(jnp.int32)
            p_i = row[0, 2].astype(jnp.int32)
            icf = row[0, 3]
            jcf = row[0, 4]
            rr0 = row[0, 5]
            rr1 = row[0, 6]
            rr2 = row[0, 7]
            rr3 = row[0, 8]
            sv = row[0, 9]
            clsi = row[0, 10].astype(jnp.int32)
            depv = row[0, 11]
            orow = full_ref[0, a_i, pl.ds(p_i, 1), :]
            chi = lax.broadcasted_iota(jnp.int32, (1, _NCH), 1)
            is23 = (chi == 2) | (chi == 3)
            outv = jnp.where(is23, orow, jax.nn.sigmoid(orow))
            tvec = ((chi >= 5) & (chi < 5 + _NCLS)
                    & ((chi - 5) == clsi)).astype(f32)
            tvec = jnp.where(chi == 0, rr0, tvec)
            tvec = jnp.where(chi == 1, rr1, tvec)
            tvec = jnp.where(chi == 2, rr2, tvec)
            tvec = jnp.where(chi == 3, rr3, tvec)
            tvec = jnp.where(chi == 4, 1.0, tvec)
            tvec = jnp.where(chi == _NCH - 1, depv, tvec)
            pc = jnp.clip(outv, 1e-7, 1.0 - 1e-7)
            bce = -(tvec * jnp.log(pc) + (1.0 - tvec) * jnp.log(1.0 - pc))
            sq = outv - tvec
            contrib = jnp.where(chi <= 1, bce * sv * sv,
                      jnp.where(is23, (sq * sv) ** 2 * 0.5,
                      jnp.where(chi == 4, bce,
                      jnp.where(chi == _NCH - 1, 0.1 * sq * sq * 0.5,
                                bce - c0))))
            bx = outv[0, 0] + icf
            by = outv[0, 1] + jcf
            aw_c = jnp.where(a_i == 0, mw[0],
                             jnp.where(a_i == 1, mw[1], mw[2]))
            ah_c = jnp.where(a_i == 0, mh[0],
                             jnp.where(a_i == 1, mh[1], mh[2]))
            bw = jnp.exp(orow[0, 2]) * aw_c
            bh = jnp.exp(orow[0, 3]) * ah_c
            ctlx = jnp.maximum(bx - bw * 0.5, t_lo_x)
            ctly = jnp.maximum(by - bh * 0.5, t_lo_y)
            cbrx = jnp.minimum(bx + bw * 0.5, t_hi_x)
            cbry = jnp.minimum(by + bh * 0.5, t_hi_y)
            cen = ((ctlx < cbrx) & (ctly < cbry)).astype(f32)
            cai = (cbrx - ctlx) * (cbry - ctly) * cen
            cu = (bw * bh + twth - cai) + 1e-16
            pbc = jnp.any(valid & (cai > 0.5 * cu))
            omc = 1.0 - pbc.astype(f32)
            p4b = jnp.clip(outv[0, 4] * omc, 1e-7, 1.0 - 1e-7)
            old_obj = -jnp.log(1.0 - p4b)
            return a + wflag * (jnp.sum(contrib) - old_obj)

        acc = lax.fori_loop(0, nmax, corr, acc)
        out_ref[0, :, :] = jnp.broadcast_to(acc, (1, 1))

    return pl.pallas_call(
        body,
        grid=(B,),
        in_specs=[
            pl.BlockSpec((1, 5, 1, P), lambda b: (b, 0, 0, 0)),
            pl.BlockSpec((1, 3, HW, _NCH), lambda b: (b, 0, 0, 0)),
            pl.BlockSpec((1, 6, nmax, 1), lambda b: (b, 0, 0, 0)),
        ],
        out_specs=pl.BlockSpec((1, 1, 1), lambda b: (b, 0, 0)),
        out_shape=jax.ShapeDtypeStruct((B, 1, 1), jnp.float32),
        scratch_shapes=[pltpu.VMEM((64, 128), jnp.float32)],
    )


def kernel(out0, out1, out2, labels):
    B = out0.shape[0]
    nmax = labels.shape[1]
    lab_t = labels.transpose(0, 2, 1).reshape(B, 6, nmax, 1)
    total = jnp.zeros((), jnp.float32)
    for oid, out in enumerate([out0, out1, out2]):
        H, W = out.shape[2], out.shape[3]
        x = out.reshape(B, 3, _NCH, H * W)
        head = x[:, :, :5, :].transpose(0, 2, 1, 3).reshape(B, 5, 1, 3 * H * W)
        full = x.transpose(0, 1, 3, 2)
        psum = _make_scale_kernel(oid, B, H, W, nmax)(head, full, lab_t)
        total = total + jnp.sum(psum)
    return total


# speedup vs baseline: 119.0619x; 3.2434x over previous
import math

import jax
import jax.numpy as jnp
import numpy as np
from jax import lax
from jax.experimental import pallas as pl
from jax.experimental.pallas import tpu as pltpu

_STRIDES = [8, 16, 32]
_ANCHORS = [[12, 16], [19, 36], [40, 28], [36, 75], [76, 55], [72, 146],
            [142, 110], [192, 243], [459, 401]]
_NCLS = 80
_NCH = 86


def _log2(n):
    return int(n).bit_length() - 1


def _atan(x):
    sgn = jnp.sign(x)
    t = jnp.abs(x)
    big = t > 2.414213562373095
    mid = t > 0.4142135623730951
    safe_t = jnp.where(big, t, 1.0)
    z_arg = jnp.where(big, -1.0 / safe_t,
                      jnp.where(mid, (t - 1.0) / (t + 1.0), t))
    z2 = z_arg * z_arg
    p = (((8.05374449538e-2 * z2 - 1.38776856032e-1) * z2
          + 1.99777106478e-1) * z2 - 3.33329491539e-1) * z2 * z_arg + z_arg
    res = jnp.where(big, math.pi / 2 + p,
                    jnp.where(mid, math.pi / 4 + p, p))
    return sgn * res


def _make_scale_kernel(oid, B, H, W, nmax):
    stride = float(_STRIDES[oid])
    HW = H * W
    P = 3 * HW
    CH = 512 if HW % 512 == 0 else HW
    log2_w = _log2(W)
    aw_all = [a[0] / stride for a in _ANCHORS]
    ah_all = [a[1] / stride for a in _ANCHORS]
    atan_ref = [float(np.arctan(np.float32(aw_all[k]) /
                                (np.float32(ah_all[k]) + np.float32(1e-16))))
                for k in range(9)]
    mw = [aw_all[3 * oid + j] for j in range(3)]
    mh = [ah_all[3 * oid + j] for j in range(3)]
    f32 = jnp.float32

    def body(x_ref, lab_ref, out_ref):
        x1 = lab_ref[0, 0]
        y1 = lab_ref[0, 1]
        x2 = lab_ref[0, 2]
        y2 = lab_ref[0, 3]
        cls = lab_ref[0, 4]
        dep = lab_ref[0, 5]
        valid = (x1 + y1 + x2 + y2 + cls + dep) > 0.0
        tx = (x2 + x1) / (stride * 2.0)
        ty = (y2 + y1) / (stride * 2.0)
        tw = (x2 - x1) / stride
        th = (y2 - y1) / stride

        area_a = tw * th
        atan_a = _atan(tw / (th + 1e-16))
        best_iou = jnp.full_like(tw, -jnp.inf)
        best_all = jnp.zeros(tw.shape, jnp.int32)
        for k in range(9):
            rw = aw_all[k]
            rh = ah_all[k]
            brx = jnp.minimum(tw, rw)
            bry = jnp.minimum(th, rh)
            en = ((0.0 < brx) & (0.0 < bry)).astype(f32)
            ai = brx * bry * en
            iou = ai / (area_a + (rw * rh) - ai + 1e-16)
            cbx = jnp.maximum(tw, rw)
            cby = jnp.maximum(th, rh)
            c2 = cbx * cbx + cby * cby + 1e-16
            rho2 = (tw - rw) ** 2 / 4.0 + (th - rh) ** 2 / 4.0
            dv = atan_a - atan_ref[k]
            v = (4.0 / math.pi ** 2) * dv * dv
            alpha = v / (1.0 - iou + v + 1e-16)
            ciou = iou - (rho2 / c2 + v * alpha)
            upd = ciou > best_iou
            best_iou = jnp.where(upd, ciou, best_iou)
            best_all = jnp.where(upd, k, best_all)
        best = best_all % 3
        use = valid & ((best_all // 3) == oid)
        has_b = jnp.any(use)

        ti = tx.astype(jnp.int32)
        tj = ty.astype(jnp.int32)
        inb = (ti >= 0) & (tj >= 0) & (ti < W) & (tj < H)
        cond = use & inb
        ic = jnp.clip(ti, 0, W - 1)
        jc = jnp.clip(tj, 0, H - 1)
        key = (best * HW + jc * W + ic).astype(f32)

        ii = lax.broadcasted_iota(jnp.int32, (nmax, nmax), 0)
        jj = lax.broadcasted_iota(jnp.int32, (nmax, nmax), 1)
        key_row0 = jnp.transpose(key)
        cond_row0 = jnp.transpose(cond.astype(f32))
        ov = (jj > ii) & (key == key_row0) & (cond_row0 > 0.0)
        overwritten = jnp.any(ov, axis=1, keepdims=True)
        winner = (cond & (~overwritten)).astype(f32)

        ic_f = ic.astype(f32)
        jc_f = jc.astype(f32)
        r0 = tx - ic_f
        r1 = ty - jc_f
        aw_b = jnp.where(best == 0, mw[0], jnp.where(best == 1, mw[1], mw[2]))
        ah_b = jnp.where(best == 0, mh[0], jnp.where(best == 1, mh[1], mh[2]))
        r2 = jnp.log(tw / aw_b + 1e-16)
        r3 = jnp.log(th / ah_b + 1e-16)
        scale_v = jnp.sqrt(jnp.maximum(2.0 - tw * th / float(W * H), 1e-8))
        q2 = jnp.where(winner > 0.0, key, -1.0)

        packed = jnp.concatenate(
            [q2, winner, best.astype(f32), ic_f, jc_f, r0, r1, r2, r3,
             scale_v, cls, dep], axis=1)
        packed_t = jnp.transpose(packed)
        q2_row = packed_t[0:1, :]
        winner_row = packed_t[1:2, :]
        best_row = packed_t[2:3, :]
        ic_row = packed_t[3:4, :]
        jc_row = packed_t[4:5, :]
        r0_row = packed_t[5:6, :]
        r1_row = packed_t[6:7, :]
        r2_row = packed_t[7:8, :]
        r3_row = packed_t[8:9, :]
        sv_row = packed_t[9:10, :]
        cls_row = packed_t[10:11, :].astype(jnp.int32)
        dep_row = packed_t[11:12, :]

        c0 = -jnp.log(1.0 - jnp.clip(jnp.zeros((), f32), 1e-7, 1.0 - 1e-7))

        t_lo_x = tx - tw * 0.5
        t_lo_y = ty - th * 0.5
        t_hi_x = tx + tw * 0.5
        t_hi_y = ty + th * 0.5
        twth = tw * th

        acc = jnp.zeros((), f32)
        for a in range(3):
            for cs in range(0, HW, CH):
                o0 = x_ref[0, a, 0:1, cs:cs + CH]
                o1 = x_ref[0, a, 1:2, cs:cs + CH]
                o2 = x_ref[0, a, 2:3, cs:cs + CH]
                o3 = x_ref[0, a, 3:4, cs:cs + CH]
                o4 = x_ref[0, a, 4:5, cs:cs + CH]
                hw = cs + lax.broadcasted_iota(jnp.int32, (1, CH), 1)
                gy = (hw >> log2_w).astype(f32)
                gx = (hw & (W - 1)).astype(f32)
                px = jax.nn.sigmoid(o0) + gx
                py = jax.nn.sigmoid(o1) + gy
                pw = jnp.exp(o2) * mw[a]
                ph = jnp.exp(o3) * mh[a]
                p_lo_x = px - pw * 0.5
                p_lo_y = py - ph * 0.5
                p_hi_x = px + pw * 0.5
                p_hi_y = py + ph * 0.5
                pwph = pw * ph
                tlx = jnp.maximum(p_lo_x, t_lo_x)
                tly = jnp.maximum(p_lo_y, t_lo_y)
                brx = jnp.minimum(p_hi_x, t_hi_x)
                bry = jnp.minimum(p_hi_y, t_hi_y)
                en = ((tlx < brx) & (tly < bry)).astype(f32)
                ai = (brx - tlx) * (bry - tly) * en
                u = (pwph + twth - ai) + 1e-16
                hit = valid & (ai > 0.5 * u)
                pb = jnp.any(hit, axis=0, keepdims=True)
                om = jnp.where(has_b, 1.0 - pb.astype(f32), 1.0)
                p4 = jnp.clip(jax.nn.sigmoid(o4) * om, 1e-7, 1.0 - 1e-7)
                acc = acc + jnp.sum(-jnp.log(1.0 - p4))
        acc = acc + c0 * float(_NCLS * P)

        rows_t = jnp.zeros((_NCH, nmax), f32)
        for a in range(3):
            pos_f = lax.broadcasted_iota(jnp.int32, (HW, nmax), 0).astype(f32)
            sel = jnp.where(pos_f + float(a * HW) == q2_row, 1.0, 0.0)
            rows_t = rows_t + lax.dot_general(
                x_ref[0, a], sel, (((1,), (0,)), ((), ())),
                precision=lax.Precision.HIGHEST,
                preferred_element_type=f32)

        chi = lax.broadcasted_iota(jnp.int32, (_NCH, 1), 0)
        is23 = (chi == 2) | (chi == 3)
        outv = jnp.where(is23, rows_t, jax.nn.sigmoid(rows_t))
        tvec = ((chi >= 5) & (chi < 5 + _NCLS)
                & ((chi - 5) == cls_row)).astype(f32)
        tvec = jnp.where(chi == 0, r0_row, tvec)
        tvec = jnp.where(chi == 1, r1_row, tvec)
        tvec = jnp.where(chi == 2, r2_row, tvec)
        tvec = jnp.where(chi == 3, r3_row, tvec)
        tvec = jnp.where(chi == 4, 1.0, tvec)
        tvec = jnp.where(chi == _NCH - 1, dep_row, tvec)
        pc = jnp.clip(outv, 1e-7, 1.0 - 1e-7)
        bce = -(tvec * jnp.log(pc) + (1.0 - tvec) * jnp.log(1.0 - pc))
        sq = outv - tvec
        contrib = jnp.where(chi <= 1, bce * sv_row * sv_row,
                  jnp.where(is23, (sq * sv_row) ** 2 * 0.5,
                  jnp.where(chi == 4, bce,
                  jnp.where(chi == _NCH - 1, 0.1 * sq * sq * 0.5,
                            bce - c0))))
        cell_sum = jnp.sum(contrib, axis=0, keepdims=True)

        bx = outv[0:1, :] + ic_row
        by = outv[1:2, :] + jc_row
        aw_c = jnp.where(best_row == 0.0, mw[0],
                         jnp.where(best_row == 1.0, mw[1], mw[2]))
        ah_c = jnp.where(best_row == 0.0, mh[0],
                         jnp.where(best_row == 1.0, mh[1], mh[2]))
        bw = jnp.exp(rows_t[2:3, :]) * aw_c
        bh = jnp.exp(rows_t[3:4, :]) * ah_c
        ctlx = jnp.maximum(bx - bw * 0.5, t_lo_x)
        ctly = jnp.maximum(by - bh * 0.5, t_lo_y)
        cbrx = jnp.minimum(bx + bw * 0.5, t_hi_x)
        cbry = jnp.minimum(by + bh * 0.5, t_hi_y)
        cen = ((ctlx < cbrx) & (ctly < cbry)).astype(f32)
        cai = (cbrx - ctlx) * (cbry - ctly) * cen
        cu = (bw * bh + twth - cai) + 1e-16
        pbc = jnp.any(valid & (cai > 0.5 * cu), axis=0, keepdims=True)
        omc = 1.0 - pbc.astype(f32)
        p4b = jnp.clip(outv[4:5, :] * omc, 1e-7, 1.0 - 1e-7)
        old_obj = -jnp.log(1.0 - p4b)
        acc = acc + jnp.sum(winner_row * (cell_sum - old_obj))
        out_ref[0, :, :] = jnp.broadcast_to(acc, (1, 1))

    return pl.pallas_call(
        body,
        grid=(B,),
        in_specs=[
            pl.BlockSpec((1, 3, _NCH, HW), lambda b: (b, 0, 0, 0)),
            pl.BlockSpec((1, 6, nmax, 1), lambda b: (b, 0, 0, 0)),
        ],
        out_specs=pl.BlockSpec((1, 1, 1), lambda b: (b, 0, 0)),
        out_shape=jax.ShapeDtypeStruct((B, 1, 1), jnp.float32),
    )


def kernel(out0, out1, out2, labels):
    B = out0.shape[0]
    nmax = labels.shape[1]
    lab_t = labels.transpose(0, 2, 1).reshape(B, 6, nmax, 1)
    total = jnp.zeros((), jnp.float32)
    for oid, out in enumerate([out0, out1, out2]):
        H, W = out.shape[2], out.shape[3]
        x = out.reshape(B, 3, _NCH, H * W)
        psum = _make_scale_kernel(oid, B, H, W, nmax)(x, lab_t)
        total = total + jnp.sum(psum)
    return total


# speedup vs baseline: 147.9515x; 1.2426x over previous
import math

import jax
import jax.numpy as jnp
import numpy as np
from jax import lax
from jax.experimental import pallas as pl
from jax.experimental.pallas import tpu as pltpu

_STRIDES = [8, 16, 32]
_ANCHORS = [[12, 16], [19, 36], [40, 28], [36, 75], [76, 55], [72, 146],
            [142, 110], [192, 243], [459, 401]]
_NCLS = 80
_NCH = 86
_BIG = 1e30


def _log2(n):
    return int(n).bit_length() - 1


def _atan(x):
    sgn = jnp.sign(x)
    t = jnp.abs(x)
    big = t > 2.414213562373095
    mid = t > 0.4142135623730951
    safe_t = jnp.where(big, t, 1.0)
    z_arg = jnp.where(big, -1.0 / safe_t,
                      jnp.where(mid, (t - 1.0) / (t + 1.0), t))
    z2 = z_arg * z_arg
    p = (((8.05374449538e-2 * z2 - 1.38776856032e-1) * z2
          + 1.99777106478e-1) * z2 - 3.33329491539e-1) * z2 * z_arg + z_arg
    res = jnp.where(big, math.pi / 2 + p,
                    jnp.where(mid, math.pi / 4 + p, p))
    return sgn * res


def _scale_loss(x_ref, lab_ref, oid, H, W, nmax):
    f32 = jnp.float32
    stride = float(_STRIDES[oid])
    HW = H * W
    P = 3 * HW
    CH = 512 if HW % 512 == 0 else HW
    log2_w = _log2(W)
    aw_all = [a[0] / stride for a in _ANCHORS]
    ah_all = [a[1] / stride for a in _ANCHORS]
    atan_ref = [float(np.arctan(np.float32(aw_all[k]) /
                                (np.float32(ah_all[k]) + np.float32(1e-16))))
                for k in range(9)]
    mw = [aw_all[3 * oid + j] for j in range(3)]
    mh = [ah_all[3 * oid + j] for j in range(3)]

    x1 = lab_ref[0, 0]
    y1 = lab_ref[0, 1]
    x2 = lab_ref[0, 2]
    y2 = lab_ref[0, 3]
    cls = lab_ref[0, 4]
    dep = lab_ref[0, 5]
    valid = (x1 + y1 + x2 + y2 + cls + dep) > 0.0
    tx = (x2 + x1) / (stride * 2.0)
    ty = (y2 + y1) / (stride * 2.0)
    tw = (x2 - x1) / stride
    th = (y2 - y1) / stride

    area_a = tw * th
    atan_a = _atan(tw / (th + 1e-16))
    best_iou = jnp.full_like(tw, -jnp.inf)
    best_all = jnp.zeros(tw.shape, jnp.int32)
    for k in range(9):
        rw = aw_all[k]
        rh = ah_all[k]
        brx = jnp.minimum(tw, rw)
        bry = jnp.minimum(th, rh)
        en = ((0.0 < brx) & (0.0 < bry)).astype(f32)
        ai = brx * bry * en
        iou = ai / (area_a + (rw * rh) - ai + 1e-16)
        cbx = jnp.maximum(tw, rw)
        cby = jnp.maximum(th, rh)
        c2 = cbx * cbx + cby * cby + 1e-16
        rho2 = (tw - rw) ** 2 / 4.0 + (th - rh) ** 2 / 4.0
        dv = atan_a - atan_ref[k]
        v = (4.0 / math.pi ** 2) * dv * dv
        alpha = v / (1.0 - iou + v + 1e-16)
        ciou = iou - (rho2 / c2 + v * alpha)
        upd = ciou > best_iou
        best_iou = jnp.where(upd, ciou, best_iou)
        best_all = jnp.where(upd, k, best_all)
    best = best_all % 3
    use = valid & ((best_all // 3) == oid)
    has_b = jnp.any(use)

    ti = tx.astype(jnp.int32)
    tj = ty.astype(jnp.int32)
    inb = (ti >= 0) & (tj >= 0) & (ti < W) & (tj < H)
    cond = use & inb
    ic = jnp.clip(ti, 0, W - 1)
    jc = jnp.clip(tj, 0, H - 1)
    key = (best * HW + jc * W + ic).astype(f32)

    ii = lax.broadcasted_iota(jnp.int32, (nmax, nmax), 0)
    jj = lax.broadcasted_iota(jnp.int32, (nmax, nmax), 1)
    key_row0 = jnp.transpose(key)
    cond_row0 = jnp.transpose(cond.astype(f32))
    ov = (jj > ii) & (key == key_row0) & (cond_row0 > 0.0)
    overwritten = jnp.any(ov, axis=1, keepdims=True)
    winner = (cond & (~overwritten)).astype(f32)

    ic_f = ic.astype(f32)
    jc_f = jc.astype(f32)
    r0 = tx - ic_f
    r1 = ty - jc_f
    aw_b = jnp.where(best == 0, mw[0], jnp.where(best == 1, mw[1], mw[2]))
    ah_b = jnp.where(best == 0, mh[0], jnp.where(best == 1, mh[1], mh[2]))
    r2 = jnp.log(tw / aw_b + 1e-16)
    r3 = jnp.log(th / ah_b + 1e-16)
    scale_v = jnp.sqrt(jnp.maximum(2.0 - tw * th / float(W * H), 1e-8))
    q2 = jnp.where(winner > 0.0, key, -1.0)

    packed = jnp.concatenate(
        [q2, winner, best.astype(f32), ic_f, jc_f, r0, r1, r2, r3,
         scale_v, cls, dep], axis=1)
    packed_t = jnp.transpose(packed)
    q2_row = packed_t[0:1, :]
    winner_row = packed_t[1:2, :]
    best_row = packed_t[2:3, :]
    ic_row = packed_t[3:4, :]
    jc_row = packed_t[4:5, :]
    r0_row = packed_t[5:6, :]
    r1_row = packed_t[6:7, :]
    r2_row = packed_t[7:8, :]
    r3_row = packed_t[8:9, :]
    sv_row = packed_t[9:10, :]
    cls_row = packed_t[10:11, :].astype(jnp.int32)
    dep_row = packed_t[11:12, :]

    c0 = -jnp.log(1.0 - jnp.clip(jnp.zeros((), f32), 1e-7, 1.0 - 1e-7))

    t_lo_x = jnp.where(valid, tx - tw * 0.5, _BIG)
    t_lo_y = jnp.where(valid, ty - th * 0.5, _BIG)
    t_hi_x = jnp.where(valid, tx + tw * 0.5, -_BIG)
    t_hi_y = jnp.where(valid, ty + th * 0.5, -_BIG)
    tarea = tw * th + 1e-16

    acc = jnp.zeros((), f32)
    for cs in range(0, HW, CH):
        hw = cs + lax.broadcasted_iota(jnp.int32, (1, CH), 1)
        gy = (hw >> log2_w).astype(f32)
        gx = (hw & (W - 1)).astype(f32)
        for a in range(3):
            o0 = x_ref[0, a, 0:1, cs:cs + CH]
            o1 = x_ref[0, a, 1:2, cs:cs + CH]
            o2 = x_ref[0, a, 2:3, cs:cs + CH]
            o3 = x_ref[0, a, 3:4, cs:cs + CH]
            o4 = x_ref[0, a, 4:5, cs:cs + CH]
            px = jax.nn.sigmoid(o0) + gx
            py = jax.nn.sigmoid(o1) + gy
            pw = jnp.exp(o2) * mw[a]
            ph = jnp.exp(o3) * mh[a]
            parea = pw * ph
            tlx = jnp.maximum(px - pw * 0.5, t_lo_x)
            tly = jnp.maximum(py - ph * 0.5, t_lo_y)
            brx = jnp.minimum(px + pw * 0.5, t_hi_x)
            bry = jnp.minimum(py + ph * 0.5, t_hi_y)
            ai3 = jnp.maximum(brx - tlx, 0.0) * jnp.maximum(bry - tly, 0.0) * 3.0
            hit = ai3 > parea + tarea
            pb = jnp.any(hit, axis=0, keepdims=True)
            om = jnp.where(has_b, 1.0 - pb.astype(f32), 1.0)
            p4 = jnp.clip(jax.nn.sigmoid(o4) * om, 1e-7, 1.0 - 1e-7)
            acc = acc + jnp.sum(-jnp.log(1.0 - p4))
    acc = acc + c0 * float(_NCLS * P)

    rows_t = jnp.zeros((_NCH, nmax), f32)
    for a in range(3):
        pos_f = lax.broadcasted_iota(jnp.int32, (HW, nmax), 0).astype(f32)
        sel = jnp.where(pos_f + float(a * HW) == q2_row,
                        1.0, 0.0).astype(jnp.bfloat16)
        slab = x_ref[0, a]
        hi = slab.astype(jnp.bfloat16)
        lo = (slab - hi.astype(f32)).astype(jnp.bfloat16)
        for part in (hi, lo):
            rows_t = rows_t + lax.dot_general(
                part, sel, (((1,), (0,)), ((), ())),
                preferred_element_type=f32)

    chi = lax.broadcasted_iota(jnp.int32, (_NCH, 1), 0)
    is23 = (chi == 2) | (chi == 3)
    outv = jnp.where(is23, rows_t, jax.nn.sigmoid(rows_t))
    tvec = ((chi >= 5) & (chi < 5 + _NCLS)
            & ((chi - 5) == cls_row)).astype(f32)
    tvec = jnp.where(chi == 0, r0_row, tvec)
    tvec = jnp.where(chi == 1, r1_row, tvec)
    tvec = jnp.where(chi == 2, r2_row, tvec)
    tvec = jnp.where(chi == 3, r3_row, tvec)
    tvec = jnp.where(chi == 4, 1.0, tvec)
    tvec = jnp.where(chi == _NCH - 1, dep_row, tvec)
    pc = jnp.clip(outv, 1e-7, 1.0 - 1e-7)
    bce = -(tvec * jnp.log(pc) + (1.0 - tvec) * jnp.log(1.0 - pc))
    sq = outv - tvec
    contrib = jnp.where(chi <= 1, bce * sv_row * sv_row,
              jnp.where(is23, (sq * sv_row) ** 2 * 0.5,
              jnp.where(chi == 4, bce,
              jnp.where(chi == _NCH - 1, 0.1 * sq * sq * 0.5,
                        bce - c0))))
    cell_sum = jnp.sum(contrib, axis=0, keepdims=True)

    bx = outv[0:1, :] + ic_row
    by = outv[1:2, :] + jc_row
    aw_c = jnp.where(best_row == 0.0, mw[0],
                     jnp.where(best_row == 1.0, mw[1], mw[2]))
    ah_c = jnp.where(best_row == 0.0, mh[0],
                     jnp.where(best_row == 1.0, mh[1], mh[2]))
    bw = jnp.exp(rows_t[2:3, :]) * aw_c
    bh = jnp.exp(rows_t[3:4, :]) * ah_c
    ctlx = jnp.maximum(bx - bw * 0.5, t_lo_x)
    ctly = jnp.maximum(by - bh * 0.5, t_lo_y)
    cbrx = jnp.minimum(bx + bw * 0.5, t_hi_x)
    cbry = jnp.minimum(by + bh * 0.5, t_hi_y)
    cai3 = jnp.maximum(cbrx - ctlx, 0.0) * jnp.maximum(cbry - ctly, 0.0) * 3.0
    pbc = jnp.any(cai3 > bw * bh + tarea, axis=0, keepdims=True)
    omc = 1.0 - pbc.astype(f32)
    p4b = jnp.clip(outv[4:5, :] * omc, 1e-7, 1.0 - 1e-7)
    old_obj = -jnp.log(1.0 - p4b)
    return acc + jnp.sum(winner_row * (cell_sum - old_obj))


def _make_kernel(B, shapes, nmax):
    def body(x0_ref, x1_ref, x2_ref, lab_ref, out_ref):
        x_refs = [x0_ref, x1_ref, x2_ref]
        total = jnp.zeros((), jnp.float32)
        for oid, (H, W) in enumerate(shapes):
            total = total + _scale_loss(x_refs[oid], lab_ref, oid, H, W, nmax)
        b = pl.program_id(0)

        @pl.when(b == 0)
        def _():
            out_ref[0, :, :] = jnp.broadcast_to(total, (1, 1))

        @pl.when(b > 0)
        def _():
            out_ref[0, :, :] = out_ref[0, :, :] + total

    specs = [pl.BlockSpec((1, 3, _NCH, h * w), lambda b: (b, 0, 0, 0))
             for (h, w) in shapes]
    specs.append(pl.BlockSpec((1, 6, nmax, 1), lambda b: (b, 0, 0, 0)))
    return pl.pallas_call(
        body,
        grid=(B,),
        in_specs=specs,
        out_specs=pl.BlockSpec((1, 1, 1), lambda b: (0, 0, 0)),
        out_shape=jax.ShapeDtypeStruct((1, 1, 1), jnp.float32),
    )


def kernel(out0, out1, out2, labels):
    B = out0.shape[0]
    nmax = labels.shape[1]
    lab_t = labels.transpose(0, 2, 1).reshape(B, 6, nmax, 1)
    shapes = [(o.shape[2], o.shape[3]) for o in (out0, out1, out2)]
    xs = [o.reshape(B, 3, _NCH, o.shape[2] * o.shape[3])
          for o in (out0, out1, out2)]
    res = _make_kernel(B, shapes, nmax)(xs[0], xs[1], xs[2], lab_t)
    return res[0, 0, 0]
